# baseline (speedup 1.0000x reference)
import jax
import jax.numpy as jnp
from jax import lax
from jax.experimental import pallas as pl
from jax.experimental.pallas import tpu as pltpu
from jax.experimental.pallas import tpu_sc as plsc

B, S, K = 4, 2048, 768
N_CLASSES = 42
NC, NS, L = 2, 16, 16

KTC = 512
KSC = K - KTC
NFC = 8
FC = KSC // NFC
NV = FC // L
TCHUNK = 128
NCHUNK = S // TCHUNK

RG = S // 8


def _sc_body(x_hbm, m1_hbm, m2_hbm, out1_hbm, out2_hbm,
             xbuf0, xbuf1, m1buf, m2buf, res1, res2, sem0, sem1):
    wid = lax.axis_index("s") * NC + lax.axis_index("c")
    bi = wid // NFC
    fc = wid % NFC
    f0 = KTC + fc * FC

    def start_fetch(ci, buf, sem):
        return pltpu.async_copy(
            x_hbm.at[bi, pl.ds(ci * TCHUNK, TCHUNK), pl.ds(f0, FC)], buf, sem)

    pltpu.sync_copy(m1_hbm.at[bi], m1buf)
    pltpu.sync_copy(m2_hbm.at[bi], m2buf)
    start_fetch(0, xbuf0, sem0)
    start_fetch(1, xbuf1, sem1)

    neg = jnp.full((L,), -jnp.inf, jnp.float32)
    for j in range(NV):
        res1[pl.ds(j * L, L)] = neg
        res2[pl.ds(j * L, L)] = neg

    def chunk_compute(ci, buf):
        cm = [neg] * (2 * NV)
        for g in range(TCHUNK // L):
            base = g * L
            m1v = m1buf[pl.ds(ci * TCHUNK + base, L)]
            m2v = m2buf[pl.ds(ci * TCHUNK + base, L)]
            bv1 = jnp.where(m1v > 0, 0.0, -jnp.inf).astype(jnp.float32)
            bv2 = jnp.where(m2v > 0, 0.0, -jnp.inf).astype(jnp.float32)
            for k in range(L):
                t = base + k
                b1 = bv1[k]
                b2 = bv2[k]
                for j in range(NV):
                    v = buf[t, pl.ds(j * L, L)]
                    cm[j] = jnp.maximum(cm[j], v + b1)
                    cm[NV + j] = jnp.maximum(cm[NV + j], v + b2)
        for j in range(NV):
            sl = pl.ds(j * L, L)
            res1[sl] = jnp.maximum(res1[sl], cm[j])
            res2[sl] = jnp.maximum(res2[sl], cm[NV + j])

    def pair_body(g, carry):
        c0 = 2 * g
        pltpu.make_async_copy(
            x_hbm.at[bi, pl.ds(0, TCHUNK), pl.ds(f0, FC)], xbuf0, sem0).wait()
        chunk_compute(c0, xbuf0)
        start_fetch(jnp.minimum(c0 + 2, NCHUNK - 2), xbuf0, sem0)
        pltpu.make_async_copy(
            x_hbm.at[bi, pl.ds(0, TCHUNK), pl.ds(f0, FC)], xbuf1, sem1).wait()
        chunk_compute(c0 + 1, xbuf1)
        start_fetch(jnp.minimum(c0 + 3, NCHUNK - 1), xbuf1, sem1)
        return carry

    lax.fori_loop(0, NCHUNK // 2, pair_body, 0)

    pltpu.make_async_copy(
        x_hbm.at[bi, pl.ds(0, TCHUNK), pl.ds(f0, FC)], xbuf0, sem0).wait()
    pltpu.make_async_copy(
        x_hbm.at[bi, pl.ds(0, TCHUNK), pl.ds(f0, FC)], xbuf1, sem1).wait()

    pltpu.sync_copy(res1, out1_hbm.at[bi, pl.ds(fc * FC, FC)])
    pltpu.sync_copy(res2, out2_hbm.at[bi, pl.ds(fc * FC, FC)])


def _sc_pool(x, m1i, m2i):
    f32 = jnp.float32
    return pl.kernel(
        _sc_body,
        out_type=(jax.ShapeDtypeStruct((B, KSC), f32),
                  jax.ShapeDtypeStruct((B, KSC), f32)),
        mesh=plsc.VectorSubcoreMesh(core_axis_name="c", subcore_axis_name="s",
                                    num_cores=NC, num_subcores=NS),
        compiler_params=pltpu.CompilerParams(use_tc_tiling_on_sc=False),
        scratch_types=[
            pltpu.VMEM((TCHUNK, FC), f32),
            pltpu.VMEM((TCHUNK, FC), f32),
            pltpu.VMEM((S,), jnp.int32),
            pltpu.VMEM((S,), jnp.int32),
            pltpu.VMEM((FC,), f32),
            pltpu.VMEM((FC,), f32),
            pltpu.SemaphoreType.DMA,
            pltpu.SemaphoreType.DMA,
        ],
    )(x, m1i, m2i)


def _tc_body(x_ref, m1t_ref, m2t_ref, o_ref):
    bi = pl.program_id(0)
    x = x_ref[0].reshape(RG, 8, KTC)
    b1c = m1t_ref[0].reshape(RG, 8, 1)
    b2c = m2t_ref[0].reshape(RG, 8, 1)
    e1 = jnp.max(x + b1c, axis=0)
    e2 = jnp.max(x + b2c, axis=0)
    row = pl.ds(bi, 1)
    o_ref[row, 0:KTC] = jnp.max(e1, axis=0, keepdims=True)
    o_ref[row, KTC:2 * KTC] = jnp.max(e2, axis=0, keepdims=True)


def _tc_pool(x, m1t, m2t):
    return pl.pallas_call(
        _tc_body,
        grid=(B,),
        in_specs=[
            pl.BlockSpec((1, S, KTC), lambda bi: (bi, 0, 0)),
            pl.BlockSpec((1, S, 1), lambda bi: (bi, 0, 0)),
            pl.BlockSpec((1, S, 1), lambda bi: (bi, 0, 0)),
        ],
        out_specs=pl.BlockSpec((B, 2 * KTC), lambda bi: (0, 0)),
        out_shape=jax.ShapeDtypeStruct((B, 2 * KTC), jnp.float32),
    )(x, m1t, m2t)


def _head_body(tp_ref, s1_ref, s2_ref, m1_ref, m2_ref, w_ref, b_ref, o_ref):
    c1 = jnp.sum(m1_ref[...], axis=1, keepdims=True)
    c2 = jnp.sum(m2_ref[...], axis=1, keepdims=True)
    pad1 = c1 < jnp.max(c1)
    pad2 = c2 < jnp.max(c2)
    p1 = jnp.concatenate([tp_ref[:, 0:KTC], s1_ref[...]], axis=-1)
    p2 = jnp.concatenate([tp_ref[:, KTC:2 * KTC], s2_ref[...]], axis=-1)
    p1 = jnp.where(pad1, jnp.maximum(p1, 0.0), p1)
    p2 = jnp.where(pad2, jnp.maximum(p2, 0.0), p2)
    dense = jnp.concatenate([p1, p2], axis=-1)
    logits = jnp.dot(dense, w_ref[...],
                     preferred_element_type=jnp.float32) + b_ref[...]
    logits = logits - jnp.max(logits, axis=-1, keepdims=True)
    e = jnp.exp(logits)
    o_ref[...] = e / jnp.sum(e, axis=-1, keepdims=True)


def _head(tcpool, sc1, sc2, m1i, m2i, W, b2d):
    return pl.pallas_call(
        _head_body,
        out_shape=jax.ShapeDtypeStruct((B, N_CLASSES), jnp.float32),
    )(tcpool, sc1, sc2, m1i, m2i, W, b2d)


def kernel(bert_output, e1_mask, e2_mask, W, b):
    m1i = e1_mask.astype(jnp.int32)
    m2i = e2_mask.astype(jnp.int32)
    f32 = jnp.float32
    neg = jnp.float32(-jnp.inf)
    m1t = jnp.where(e1_mask, 0.0, neg).astype(f32).reshape(B, S, 1)
    m2t = jnp.where(e2_mask, 0.0, neg).astype(f32).reshape(B, S, 1)
    sc1, sc2 = _sc_pool(bert_output, m1i, m2i)
    tcpool = _tc_pool(bert_output, m1t, m2t)
    return _head(tcpool, sc1, sc2, m1i, m2i, W, b.reshape(1, N_CLASSES))


# speedup vs baseline: 3.2395x; 3.2395x over previous
import jax
import jax.numpy as jnp
from jax import lax
from jax.experimental import pallas as pl
from jax.experimental.pallas import tpu as pltpu

B, S, K = 4, 2048, 768
N_CLASSES = 42
RG = S // 8


def _tc_body(x_ref, m1r_ref, m2r_ref, m1_ref, m2_ref, w_ref, b_ref,
             o_ref, pool_ref):
    bi = pl.program_id(0)
    neg = jnp.float32(-jnp.inf)

    x = x_ref[0].reshape(RG, 8, K)
    m1c = m1r_ref[0].reshape(RG, 8, 1)
    m2c = m2r_ref[0].reshape(RG, 8, 1)
    e1 = jnp.max(jnp.where(m1c > 0, x, neg), axis=0)
    e2 = jnp.max(jnp.where(m2c > 0, x, neg), axis=0)
    row = pl.ds(bi, 1)
    pool_ref[row, 0:K] = jnp.max(e1, axis=0, keepdims=True)
    pool_ref[row, K:2 * K] = jnp.max(e2, axis=0, keepdims=True)

    @pl.when(bi == B - 1)
    def _():
        c1 = jnp.sum(m1_ref[...], axis=1, keepdims=True)
        c2 = jnp.sum(m2_ref[...], axis=1, keepdims=True)
        pad1 = c1 < jnp.max(c1)
        pad2 = c2 < jnp.max(c2)
        p1 = pool_ref[:, 0:K]
        p2 = pool_ref[:, K:2 * K]
        p1 = jnp.where(pad1, jnp.maximum(p1, 0.0), p1)
        p2 = jnp.where(pad2, jnp.maximum(p2, 0.0), p2)
        dense = jnp.concatenate([p1, p2], axis=-1)
        logits = jnp.dot(dense, w_ref[...],
                         preferred_element_type=jnp.float32) + b_ref[...]
        logits = logits - jnp.max(logits, axis=-1, keepdims=True)
        e = jnp.exp(logits)
        o_ref[...] = e / jnp.sum(e, axis=-1, keepdims=True)


def kernel(bert_output, e1_mask, e2_mask, W, b):
    m1i = e1_mask.astype(jnp.int32)
    m2i = e2_mask.astype(jnp.int32)
    m1r = m1i.reshape(B, S, 1)
    m2r = m2i.reshape(B, S, 1)
    return pl.pallas_call(
        _tc_body,
        grid=(B,),
        in_specs=[
            pl.BlockSpec((1, S, K), lambda bi: (bi, 0, 0)),
            pl.BlockSpec((1, S, 1), lambda bi: (bi, 0, 0)),
            pl.BlockSpec((1, S, 1), lambda bi: (bi, 0, 0)),
            pl.BlockSpec((B, S), lambda bi: (0, 0)),
            pl.BlockSpec((B, S), lambda bi: (0, 0)),
            pl.BlockSpec((2 * K, N_CLASSES), lambda bi: (0, 0)),
            pl.BlockSpec((1, N_CLASSES), lambda bi: (0, 0)),
        ],
        out_specs=pl.BlockSpec((B, N_CLASSES), lambda bi: (0, 0)),
        out_shape=jax.ShapeDtypeStruct((B, N_CLASSES), jnp.float32),
        scratch_shapes=[pltpu.VMEM((B, 2 * K), jnp.float32)],
    )(bert_output, m1r, m2r, m1i, m2i, W, b.reshape(1, N_CLASSES))
